# Initial kernel scaffold; baseline (speedup 1.0000x reference)
import jax
import jax.numpy as jnp
from jax.experimental import pallas as pl


def kernel(x, edge_index, W1, b1, W2, b2, W3, b3, Wc, bc):
    raise NotImplementedError("write your pallas kernel here")


# speedup vs baseline: 10.9093x; 10.9093x over previous
import jax
import jax.numpy as jnp
from jax import lax
from jax.experimental import pallas as pl
from jax.experimental.pallas import tpu as pltpu
from jax.experimental.pallas import tpu_sc as plsc

N = 10000
E = 320000
NSUB = 16
NCORE = 2
EB = 128
E_PAD = 323584
JB = 79
NACC = 10240
RPS = NACC // NSUB
OPS = 632
OPS_LAST = N - 15 * OPS


def _sc_mesh():
    return plsc.VectorSubcoreMesh(core_axis_name="c", subcore_axis_name="s")


def _deg_body(dst32, ones_h, zr_h, out, dstv, onesv, acc, sem):
    c = lax.axis_index("c")
    s = lax.axis_index("s")
    w = c * NSUB + s
    pltpu.sync_copy(dst32.at[w], dstv)
    pltpu.sync_copy(ones_h, onesv)
    pltpu.sync_copy(zr_h, acc.at[pl.ds(s * RPS, RPS)])
    plsc.subcore_barrier()

    def step(j, carry):
        pltpu.sync_copy(onesv, acc.at[dstv.at[j]], add=True)
        return carry

    lax.fori_loop(0, JB, step, 0)
    plsc.subcore_barrier()

    @pl.when(s < NSUB - 1)
    def _():
        pltpu.sync_copy(acc.at[pl.ds(s * OPS, OPS)],
                        out.at[c, pl.ds(s * OPS, OPS)])

    @pl.when(s == NSUB - 1)
    def _():
        pltpu.sync_copy(acc.at[pl.ds(15 * OPS, OPS_LAST)],
                        out.at[c, pl.ds(15 * OPS, OPS_LAST)])


_deg_call = pl.kernel(
    _deg_body,
    out_type=jax.ShapeDtypeStruct((NCORE, N, 128), jnp.float32),
    mesh=_sc_mesh(),
    scratch_types=[
        pltpu.VMEM((JB, EB), jnp.int32),
        pltpu.VMEM((EB, 128), jnp.float32),
        pltpu.VMEM_SHARED((NACC, 128), jnp.float32),
        pltpu.SemaphoreType.DMA,
    ],
    name="sc_degree",
)


def _copy_out(acc, out_ref, s):
    @pl.when(s < NSUB - 1)
    def _():
        pltpu.sync_copy(acc.at[pl.ds(s * OPS, OPS)],
                        out_ref.at[pl.ds(s * OPS, OPS)])

    @pl.when(s == NSUB - 1)
    def _():
        pltpu.sync_copy(acc.at[pl.ds(15 * OPS, OPS_LAST)],
                        out_ref.at[pl.ds(15 * OPS, OPS_LAST)])


def _edge_pass(y_ref, src32, dst32, row, srcv, dstv, rows, acc, sem):
    pltpu.sync_copy(src32.at[row], srcv)
    pltpu.sync_copy(dst32.at[row], dstv)

    def step(j, carry):
        pltpu.async_copy(y_ref.at[srcv.at[j]], rows, sem).wait()
        pltpu.sync_copy(rows, acc.at[dstv.at[j]], add=True)
        return carry

    lax.fori_loop(0, JB, step, 0)


def _agg4_body(y0, y1, y2, y3, src32, dst32, zr_h,
               o0, o1, o2, o3, srcv, dstv, rows, acc, sem):
    c = lax.axis_index("c")
    s = lax.axis_index("s")
    ys = (y0, y1, y2, y3)
    outs = (o0, o1, o2, o3)
    for ci in range(4):
        @pl.when(c == ci // 2)
        def _(ci=ci):
            pltpu.sync_copy(zr_h, acc.at[pl.ds(s * RPS, RPS)])
            plsc.subcore_barrier()
            for p in range(2):
                _edge_pass(ys[ci], src32, dst32, 2 * s + p,
                           srcv, dstv, rows, acc, sem)
            plsc.subcore_barrier()
            _copy_out(acc, outs[ci], s)
            plsc.subcore_barrier()


def _agg1_body(y, src32, dst32, zr_h, out, srcv, dstv, rows, acc, sem):
    c = lax.axis_index("c")
    s = lax.axis_index("s")
    pltpu.sync_copy(zr_h, acc.at[pl.ds(s * RPS, RPS)])
    plsc.subcore_barrier()
    _edge_pass(y, src32, dst32, c * NSUB + s, srcv, dstv, rows, acc, sem)
    plsc.subcore_barrier()
    _copy_out(acc, out.at[c], s)


def _agg_scratch():
    return [
        pltpu.VMEM((JB, EB), jnp.int32),
        pltpu.VMEM((JB, EB), jnp.int32),
        pltpu.VMEM((EB, 128), jnp.float32),
        pltpu.VMEM_SHARED((NACC, 128), jnp.float32),
        pltpu.SemaphoreType.DMA,
    ]


_agg4 = pl.kernel(
    _agg4_body,
    out_type=[jax.ShapeDtypeStruct((N, 128), jnp.float32) for _ in range(4)],
    mesh=_sc_mesh(),
    scratch_types=_agg_scratch(),
    name="sc_agg4",
)

_agg1 = pl.kernel(
    _agg1_body,
    out_type=jax.ShapeDtypeStruct((NCORE, N, 128), jnp.float32),
    mesh=_sc_mesh(),
    scratch_types=_agg_scratch(),
    name="sc_agg1",
)


def _prep_body(dp_ref, x_ref, dinv_ref, yx_ref):
    deg = dp_ref[0, :, 0:1] + dp_ref[1, :, 0:1] + 1.0
    dinv = lax.rsqrt(deg)
    dinv_ref[...] = dinv
    yx_ref[...] = x_ref[...] * dinv


def _prep(degparts, x):
    return pl.pallas_call(
        _prep_body,
        out_shape=[jax.ShapeDtypeStruct((N, 1), jnp.float32),
                   jax.ShapeDtypeStruct((N, 128), jnp.float32)],
    )(degparts, x)


_RB = 1000


def _layer1_body(sx, yx, dinv, w1, b1, w2, y2):
    a = dinv[...] * (sx[0] + sx[1] + yx[...])
    t = jnp.dot(a, w1[...], preferred_element_type=jnp.float32) + b1[...]
    u = jnp.maximum(t, 0.0) * dinv[...]
    y2[...] = jnp.dot(u, w2[...], preferred_element_type=jnp.float32)


def _layer1(Sx, Yx, dinv, W1, b1, W2):
    g = N // _RB
    return pl.pallas_call(
        _layer1_body,
        grid=(g,),
        in_specs=[
            pl.BlockSpec((NCORE, _RB, 128), lambda i: (0, i, 0)),
            pl.BlockSpec((_RB, 128), lambda i: (i, 0)),
            pl.BlockSpec((_RB, 1), lambda i: (i, 0)),
            pl.BlockSpec((128, 1024), lambda i: (0, 0)),
            pl.BlockSpec((1, 1024), lambda i: (0, 0)),
            pl.BlockSpec((1024, 512), lambda i: (0, 0)),
        ],
        out_specs=pl.BlockSpec((_RB, 512), lambda i: (i, 0)),
        out_shape=jax.ShapeDtypeStruct((N, 512), jnp.float32),
    )(Sx, Yx, dinv, W1, b1, W2)


def _layer2_body(s2, y2, dinv, b2, w3, y3):
    a = dinv[...] * (s2[...] + y2[...]) + b2[...]
    u = jnp.maximum(a, 0.0) * dinv[...]
    y3[...] = jnp.dot(u, w3[...], preferred_element_type=jnp.float32)


def _layer2(S2, Y2, dinv, b2, W3):
    g = N // _RB
    return pl.pallas_call(
        _layer2_body,
        grid=(g,),
        in_specs=[
            pl.BlockSpec((_RB, 512), lambda i: (i, 0)),
            pl.BlockSpec((_RB, 512), lambda i: (i, 0)),
            pl.BlockSpec((_RB, 1), lambda i: (i, 0)),
            pl.BlockSpec((1, 512), lambda i: (0, 0)),
            pl.BlockSpec((512, 128), lambda i: (0, 0)),
        ],
        out_specs=pl.BlockSpec((_RB, 128), lambda i: (i, 0)),
        out_shape=jax.ShapeDtypeStruct((N, 128), jnp.float32),
    )(S2, Y2, dinv, b2, W3)


def _layer3_body(s3, y3, dinv, b3, wc, bc, out):
    h = jnp.maximum(dinv[...] * (s3[0] + s3[1] + y3[...]) + b3[...], 0.0)
    out[...] = jnp.dot(h, wc[...], preferred_element_type=jnp.float32) + bc[...]


def _layer3(S3, Y3, dinv, b3, Wc, bc):
    g = N // _RB
    return pl.pallas_call(
        _layer3_body,
        grid=(g,),
        in_specs=[
            pl.BlockSpec((NCORE, _RB, 128), lambda i: (0, i, 0)),
            pl.BlockSpec((_RB, 128), lambda i: (i, 0)),
            pl.BlockSpec((_RB, 1), lambda i: (i, 0)),
            pl.BlockSpec((1, 128), lambda i: (0, 0)),
            pl.BlockSpec((128, 10), lambda i: (0, 0)),
            pl.BlockSpec((1, 10), lambda i: (0, 0)),
        ],
        out_specs=pl.BlockSpec((_RB, 10), lambda i: (i, 0)),
        out_shape=jax.ShapeDtypeStruct((N, 10), jnp.float32),
    )(S3, Y3, dinv, b3, Wc, bc)


def kernel(x, edge_index, W1, b1, W2, b2, W3, b3, Wc, bc):
    ei = edge_index.astype(jnp.int32)
    pad = E_PAD - E
    src = jnp.concatenate([ei[0], jnp.zeros((pad,), jnp.int32)])
    dst = jnp.concatenate([ei[1], jnp.full((pad,), N, jnp.int32)])
    src32 = src.reshape(32, JB, EB)
    dst32 = dst.reshape(32, JB, EB)
    ones128 = jnp.ones((EB, 128), jnp.float32)
    zr = jnp.zeros((RPS, 128), jnp.float32)

    degparts = _deg_call(dst32, ones128, zr)
    dinv, yx = _prep(degparts, x)

    Sx = _agg1(yx, src32, dst32, zr)
    y2 = _layer1(Sx, yx, dinv, W1, b1.reshape(1, -1), W2)

    s2 = _agg4(y2[:, :128], y2[:, 128:256], y2[:, 256:384], y2[:, 384:],
               src32, dst32, zr)
    S2 = jnp.concatenate(s2, axis=1)
    y3 = _layer2(S2, y2, dinv, b2.reshape(1, -1), W3)

    S3 = _agg1(y3, src32, dst32, zr)
    return _layer3(S3, y3, dinv, b3.reshape(1, -1), Wc, bc.reshape(1, -1))


# speedup vs baseline: 13.8637x; 1.2708x over previous
import jax
import jax.numpy as jnp
from jax import lax
from jax.experimental import pallas as pl
from jax.experimental.pallas import tpu as pltpu
from jax.experimental.pallas import tpu_sc as plsc

N = 10000
E = 320000
NSUB = 16
NCORE = 2
EB = 128
E_PAD = 323584
JB = 79
NACC = 10240
RPS = NACC // NSUB
OPS = 632
OPS_LAST = N - 15 * OPS


def _sc_mesh():
    return plsc.VectorSubcoreMesh(core_axis_name="c", subcore_axis_name="s")


def _deg_body(dst32, ones_h, zr_h, out, dstv, onesv, acc, sem):
    c = lax.axis_index("c")
    s = lax.axis_index("s")
    w = c * NSUB + s
    pltpu.sync_copy(dst32.at[w], dstv)
    pltpu.sync_copy(ones_h, onesv)
    pltpu.sync_copy(zr_h, acc.at[pl.ds(s * RPS, RPS)])
    plsc.subcore_barrier()

    def step(j, carry):
        pltpu.sync_copy(onesv, acc.at[dstv.at[j]], add=True)
        return carry

    lax.fori_loop(0, JB, step, 0)
    plsc.subcore_barrier()

    @pl.when(s < NSUB - 1)
    def _():
        pltpu.sync_copy(acc.at[pl.ds(s * OPS, OPS)],
                        out.at[c, pl.ds(s * OPS, OPS)])

    @pl.when(s == NSUB - 1)
    def _():
        pltpu.sync_copy(acc.at[pl.ds(15 * OPS, OPS_LAST)],
                        out.at[c, pl.ds(15 * OPS, OPS_LAST)])


_deg_call = pl.kernel(
    _deg_body,
    out_type=jax.ShapeDtypeStruct((NCORE, N, 128), jnp.float32),
    mesh=_sc_mesh(),
    scratch_types=[
        pltpu.VMEM((JB, EB), jnp.int32),
        pltpu.VMEM((EB, 128), jnp.float32),
        pltpu.VMEM_SHARED((NACC, 128), jnp.float32),
        pltpu.SemaphoreType.DMA,
    ],
    name="sc_degree",
)


def _copy_out(acc, out_ref, s):
    @pl.when(s < NSUB - 1)
    def _():
        pltpu.sync_copy(acc.at[pl.ds(s * OPS, OPS)],
                        out_ref.at[pl.ds(s * OPS, OPS)])

    @pl.when(s == NSUB - 1)
    def _():
        pltpu.sync_copy(acc.at[pl.ds(15 * OPS, OPS_LAST)],
                        out_ref.at[pl.ds(15 * OPS, OPS_LAST)])


JH = (40, 39)


def _edge_pass(y_ref, src32, dst32, row, srcv, dstv, rows0, rows1,
               acc, sem0, sem1):
    for p, nb in enumerate(JH):
        lo = p * JH[0]
        pltpu.sync_copy(src32.at[row, pl.ds(lo, nb)], srcv.at[pl.ds(0, nb)])
        pltpu.sync_copy(dst32.at[row, pl.ds(lo, nb)], dstv.at[pl.ds(0, nb)])
        pltpu.async_copy(y_ref.at[srcv.at[0]], rows0, sem0)

        def step(j, carry):
            @pl.when(j % 2 == 0)
            def _():
                @pl.when(j < nb - 1)
                def _():
                    pltpu.async_copy(y_ref.at[srcv.at[j + 1]], rows1, sem1)
                pltpu.make_async_copy(y_ref.at[srcv.at[j]], rows0, sem0).wait()
                pltpu.sync_copy(rows0, acc.at[dstv.at[j]], add=True)

            @pl.when(j % 2 == 1)
            def _():
                @pl.when(j < nb - 1)
                def _():
                    pltpu.async_copy(y_ref.at[srcv.at[j + 1]], rows0, sem0)
                pltpu.make_async_copy(y_ref.at[srcv.at[j]], rows1, sem1).wait()
                pltpu.sync_copy(rows1, acc.at[dstv.at[j]], add=True)

            return carry

        lax.fori_loop(0, nb, step, 0)


def _agg4_body(y0, y1, y2, y3, src32, dst32, zr_h,
               o0, o1, o2, o3, srcv, dstv, rows0, rows1, acc, sem0, sem1):
    c = lax.axis_index("c")
    s = lax.axis_index("s")
    ys = (y0, y1, y2, y3)
    outs = (o0, o1, o2, o3)
    for ci in range(4):
        @pl.when(c == ci // 2)
        def _(ci=ci):
            pltpu.sync_copy(zr_h, acc.at[pl.ds(s * RPS, RPS)])
            plsc.subcore_barrier()
            for p in range(2):
                _edge_pass(ys[ci], src32, dst32, 2 * s + p,
                           srcv, dstv, rows0, rows1, acc, sem0, sem1)
            plsc.subcore_barrier()
            _copy_out(acc, outs[ci], s)
            plsc.subcore_barrier()


def _agg1_body(y, src32, dst32, zr_h, out,
               srcv, dstv, rows0, rows1, acc, sem0, sem1):
    c = lax.axis_index("c")
    s = lax.axis_index("s")
    pltpu.sync_copy(zr_h, acc.at[pl.ds(s * RPS, RPS)])
    plsc.subcore_barrier()
    _edge_pass(y, src32, dst32, c * NSUB + s,
               srcv, dstv, rows0, rows1, acc, sem0, sem1)
    plsc.subcore_barrier()
    _copy_out(acc, out.at[c], s)


def _agg_scratch():
    return [
        pltpu.VMEM((JH[0], EB), jnp.int32),
        pltpu.VMEM((JH[0], EB), jnp.int32),
        pltpu.VMEM((EB, 128), jnp.float32),
        pltpu.VMEM((EB, 128), jnp.float32),
        pltpu.VMEM_SHARED((NACC, 128), jnp.float32),
        pltpu.SemaphoreType.DMA,
        pltpu.SemaphoreType.DMA,
    ]


_agg4 = pl.kernel(
    _agg4_body,
    out_type=[jax.ShapeDtypeStruct((N, 128), jnp.float32) for _ in range(4)],
    mesh=_sc_mesh(),
    scratch_types=_agg_scratch(),
    name="sc_agg4",
)

_agg1 = pl.kernel(
    _agg1_body,
    out_type=jax.ShapeDtypeStruct((NCORE, N, 128), jnp.float32),
    mesh=_sc_mesh(),
    scratch_types=_agg_scratch(),
    name="sc_agg1",
)


def _prep_body(dp_ref, x_ref, dinv_ref, yx_ref):
    deg = dp_ref[0, :, 0:1] + dp_ref[1, :, 0:1] + 1.0
    dinv = lax.rsqrt(deg)
    dinv_ref[...] = dinv
    yx_ref[...] = x_ref[...] * dinv


def _prep(degparts, x):
    return pl.pallas_call(
        _prep_body,
        out_shape=[jax.ShapeDtypeStruct((N, 1), jnp.float32),
                   jax.ShapeDtypeStruct((N, 128), jnp.float32)],
    )(degparts, x)


_RB = 1000


def _layer1_body(sx, yx, dinv, w1, b1, w2, y2):
    a = dinv[...] * (sx[0] + sx[1] + yx[...])
    t = jnp.dot(a, w1[...], preferred_element_type=jnp.float32) + b1[...]
    u = jnp.maximum(t, 0.0) * dinv[...]
    y2[...] = jnp.dot(u, w2[...], preferred_element_type=jnp.float32)


def _layer1(Sx, Yx, dinv, W1, b1, W2):
    g = N // _RB
    return pl.pallas_call(
        _layer1_body,
        grid=(g,),
        in_specs=[
            pl.BlockSpec((NCORE, _RB, 128), lambda i: (0, i, 0)),
            pl.BlockSpec((_RB, 128), lambda i: (i, 0)),
            pl.BlockSpec((_RB, 1), lambda i: (i, 0)),
            pl.BlockSpec((128, 1024), lambda i: (0, 0)),
            pl.BlockSpec((1, 1024), lambda i: (0, 0)),
            pl.BlockSpec((1024, 512), lambda i: (0, 0)),
        ],
        out_specs=pl.BlockSpec((_RB, 512), lambda i: (i, 0)),
        out_shape=jax.ShapeDtypeStruct((N, 512), jnp.float32),
    )(Sx, Yx, dinv, W1, b1, W2)


def _layer2_body(s2, y2, dinv, b2, w3, y3):
    a = dinv[...] * (s2[...] + y2[...]) + b2[...]
    u = jnp.maximum(a, 0.0) * dinv[...]
    y3[...] = jnp.dot(u, w3[...], preferred_element_type=jnp.float32)


def _layer2(S2, Y2, dinv, b2, W3):
    g = N // _RB
    return pl.pallas_call(
        _layer2_body,
        grid=(g,),
        in_specs=[
            pl.BlockSpec((_RB, 512), lambda i: (i, 0)),
            pl.BlockSpec((_RB, 512), lambda i: (i, 0)),
            pl.BlockSpec((_RB, 1), lambda i: (i, 0)),
            pl.BlockSpec((1, 512), lambda i: (0, 0)),
            pl.BlockSpec((512, 128), lambda i: (0, 0)),
        ],
        out_specs=pl.BlockSpec((_RB, 128), lambda i: (i, 0)),
        out_shape=jax.ShapeDtypeStruct((N, 128), jnp.float32),
    )(S2, Y2, dinv, b2, W3)


def _layer3_body(s3, y3, dinv, b3, wc, bc, out):
    h = jnp.maximum(dinv[...] * (s3[0] + s3[1] + y3[...]) + b3[...], 0.0)
    out[...] = jnp.dot(h, wc[...], preferred_element_type=jnp.float32) + bc[...]


def _layer3(S3, Y3, dinv, b3, Wc, bc):
    g = N // _RB
    return pl.pallas_call(
        _layer3_body,
        grid=(g,),
        in_specs=[
            pl.BlockSpec((NCORE, _RB, 128), lambda i: (0, i, 0)),
            pl.BlockSpec((_RB, 128), lambda i: (i, 0)),
            pl.BlockSpec((_RB, 1), lambda i: (i, 0)),
            pl.BlockSpec((1, 128), lambda i: (0, 0)),
            pl.BlockSpec((128, 10), lambda i: (0, 0)),
            pl.BlockSpec((1, 10), lambda i: (0, 0)),
        ],
        out_specs=pl.BlockSpec((_RB, 10), lambda i: (i, 0)),
        out_shape=jax.ShapeDtypeStruct((N, 10), jnp.float32),
    )(S3, Y3, dinv, b3, Wc, bc)


def kernel(x, edge_index, W1, b1, W2, b2, W3, b3, Wc, bc):
    ei = edge_index.astype(jnp.int32)
    pad = E_PAD - E
    src = jnp.concatenate([ei[0], jnp.zeros((pad,), jnp.int32)])
    dst = jnp.concatenate([ei[1], jnp.full((pad,), N, jnp.int32)])
    src32 = src.reshape(32, JB, EB)
    dst32 = dst.reshape(32, JB, EB)
    ones128 = jnp.ones((EB, 128), jnp.float32)
    zr = jnp.zeros((RPS, 128), jnp.float32)

    degparts = _deg_call(dst32, ones128, zr)
    dinv, yx = _prep(degparts, x)

    Sx = _agg1(yx, src32, dst32, zr)
    y2 = _layer1(Sx, yx, dinv, W1, b1.reshape(1, -1), W2)

    s2 = _agg4(y2[:, :128], y2[:, 128:256], y2[:, 256:384], y2[:, 384:],
               src32, dst32, zr)
    S2 = jnp.concatenate(s2, axis=1)
    y3 = _layer2(S2, y2, dinv, b2.reshape(1, -1), W3)

    S3 = _agg1(y3, src32, dst32, zr)
    return _layer3(S3, y3, dinv, b3.reshape(1, -1), Wc, bc.reshape(1, -1))
